# baseline (speedup 1.0000x reference)
import jax
import jax.numpy as jnp
from jax.experimental import pallas as pl
from jax.experimental.pallas import tpu as pltpu

_DN = (((1,), (1,)), ((), ()))

_CM = 200
_NBUF = 4


def _sage_manual_kernel(adj_hbm, feat_ref, w_ref, b_ref, out_ref, bufs, sems):
    n, d = feat_ref.shape
    nc = n // _CM

    def start(i, slot):
        pltpu.make_async_copy(
            adj_hbm.at[pl.ds(i * _CM, _CM), :], bufs.at[slot], sems.at[slot]
        ).start()

    def wait(i, slot):
        pltpu.make_async_copy(
            adj_hbm.at[pl.ds(i * _CM, _CM), :], bufs.at[slot], sems.at[slot]
        ).wait()

    for s in range(_NBUF):
        start(s, s)

    def loop(i, carry):
        slot = jax.lax.rem(i, _NBUF)
        wait(i, slot)
        a = bufs[slot]
        nb = jnp.dot(a, feat_ref[...], preferred_element_type=jnp.float32)
        self_f = feat_ref[pl.ds(i * _CM, _CM), :]
        out = (
            jax.lax.dot_general(
                self_f, w_ref[:, 0:d], _DN, preferred_element_type=jnp.float32
            )
            + jax.lax.dot_general(
                nb, w_ref[:, d : 2 * d], _DN, preferred_element_type=jnp.float32
            )
            + b_ref[...]
        )
        norm = jnp.sqrt(jnp.sum(out * out, axis=1, keepdims=True))
        out_ref[pl.ds(i * _CM, _CM), :] = out / jnp.maximum(norm, 1e-12)

        @pl.when(i + _NBUF < nc)
        def _():
            start(i + _NBUF, slot)

        return carry

    jax.lax.fori_loop(0, nc, loop, 0)


def kernel(features, adj, W, b):
    n, d = features.shape
    b2 = b.reshape(1, d)
    return pl.pallas_call(
        _sage_manual_kernel,
        in_specs=[
            pl.BlockSpec(memory_space=pl.ANY),
            pl.BlockSpec(memory_space=pltpu.MemorySpace.VMEM),
            pl.BlockSpec(memory_space=pltpu.MemorySpace.VMEM),
            pl.BlockSpec(memory_space=pltpu.MemorySpace.VMEM),
        ],
        out_specs=pl.BlockSpec(memory_space=pltpu.MemorySpace.VMEM),
        out_shape=jax.ShapeDtypeStruct((n, d), jnp.float32),
        scratch_shapes=[
            pltpu.VMEM((_NBUF, _CM, 10000), jnp.float32),
            pltpu.SemaphoreType.DMA((_NBUF,)),
        ],
        compiler_params=pltpu.CompilerParams(
            vmem_limit_bytes=100 * 1024 * 1024,
        ),
    )(adj, features, W, b2)
